# baseline (speedup 1.0000x reference)
import functools

import jax
import jax.numpy as jnp
from jax import lax
from jax.experimental import pallas as pl
from jax.experimental.pallas import tpu as pltpu
from jax.experimental.pallas import tpu_sc as plsc

N = 10000
E = 320000
NHID = 128
NCLASS = 64

_B = 48
_EPT = 10080
_EPAD = 32 * _EPT
_TPB = _EPT // _B
_CHUNK = 30
_NCHUNK = _TPB // _CHUNK
_NSLOT = 5
_NPAD = 10112
_STRIPE = _NPAD // 16

_DN = lax.GatherDimensionNumbers(
    offset_dims=(), collapsed_slice_dims=(0,), start_index_map=(0,))


def _make_edge_agg(d):
  mesh = plsc.VectorSubcoreMesh(core_axis_name="c", subcore_axis_name="s")

  @functools.partial(
      pl.kernel,
      mesh=mesh,
      out_type=jax.ShapeDtypeStruct((2, _NPAD, d), jnp.float32),
      scratch_types=[
          pltpu.VMEM((_CHUNK, _B), jnp.int32),
          pltpu.VMEM((_CHUNK, _B), jnp.int32),
          pltpu.VMEM((_CHUNK * _B,), jnp.float32),
          pltpu.VMEM((_NSLOT * _B, d), jnp.float32),
          pltpu.VMEM_SHARED((_NPAD, d), jnp.float32),
          pltpu.SemaphoreType.DMA,
          pltpu.SemaphoreType.DMA,
      ],
  )
  def edge_agg(s_hbm, src_hbm, dst_hbm, adj_hbm, out_hbm,
               src_v, dst_v, adj_v, ring_v, acc, gsem, ssem):
    c = lax.axis_index("c")
    sid = lax.axis_index("s")
    w = c * 16 + sid

    def _slot(t):
      return ring_v.at[pl.ds(pl.multiple_of(lax.rem(t, _NSLOT) * _B, 16), _B)]

    def zrow(i, carry):
      for k in range(d // 16):
        ring_v[i, pl.ds(k * 16, 16)] = jnp.zeros((16,), jnp.float32)
      return carry

    lax.fori_loop(0, 80, zrow, 0)
    zcopies = [(q * 80, 80) for q in range(_STRIPE // 80)]
    zcopies.append(((_STRIPE // 80) * 80, _STRIPE - (_STRIPE // 80) * 80))
    for off, n in zcopies:
      pltpu.async_copy(ring_v.at[pl.ds(0, n)],
                       acc.at[pl.ds(sid * _STRIPE + off, n)], gsem)
    for off, n in zcopies:
      pltpu.make_async_copy(ring_v.at[pl.ds(0, n)],
                            acc.at[pl.ds(sid * _STRIPE + off, n)], gsem).wait()
    plsc.subcore_barrier()

    def _load_idx(q):
      m = w * _NCHUNK + q
      pltpu.sync_copy(src_hbm.at[m], src_v)
      pltpu.sync_copy(dst_hbm.at[m], dst_v)
      pltpu.sync_copy(adj_hbm.at[pl.ds(m * (_CHUNK * _B), _CHUNK * _B)],
                      adj_v)

    def _gather(jj, i):
      pltpu.async_copy(s_hbm.at[src_v.at[jj]], _slot(i), gsem)

    def _wait_gather(jj, i):
      pltpu.make_async_copy(s_hbm.at[src_v.at[jj]], _slot(i), gsem).wait()

    def _scatter(jj, i):
      pltpu.async_copy(_slot(i), acc.at[dst_v.at[jj]], ssem, add=True)

    def _drain_scatter():
      pltpu.make_async_copy(ring_v.at[pl.ds(0, _B)], acc.at[dst_v.at[0]],
                            ssem).wait()

    def scale_batch(t, jj):
      base = pl.multiple_of(lax.rem(t, _NSLOT) * _B, 16)

      def scale(g, c2):
        va = adj_v[pl.ds(jj * _B + g * 16, 16)]
        for l in range(16):
          e = base + g * 16 + l
          a = lax.gather(va, jnp.full((16, 1), l, jnp.int32), _DN,
                         slice_sizes=(1,),
                         mode=lax.GatherScatterMode.PROMISE_IN_BOUNDS)
          for k in range(d // 16):
            ring_v[e, pl.ds(k * 16, 16)] = ring_v[e, pl.ds(k * 16, 16)] * a
        return c2

      lax.fori_loop(0, _B // 16, scale, 0)

    _load_idx(0)
    for i0 in range(3):
      _gather(i0, i0)

    def chunk(q, carry):
      def batch(jj, carry2):
        t = q * _CHUNK + jj
        _wait_gather(jj, t)

        @pl.when(jj >= 1)
        def _():
          _drain_scatter()

        @pl.when(jnp.logical_and(jj < _CHUNK - 3, t < _TPB - 3))
        def _():
          _gather(jj + 3, t + 3)

        scale_batch(t, jj)
        _scatter(jj, t)
        return carry2

      lax.fori_loop(0, _CHUNK, batch, 0)
      _drain_scatter()

      @pl.when(q < _NCHUNK - 1)
      def _():
        _load_idx(q + 1)
        for i0 in range(3):
          _gather(i0, (q + 1) * _CHUNK + i0)

      return carry

    lax.fori_loop(0, _NCHUNK, chunk, 0)
    plsc.subcore_barrier()
    pltpu.sync_copy(acc.at[pl.ds(sid * _STRIPE, _STRIPE)],
                    out_hbm.at[c].at[pl.ds(sid * _STRIPE, _STRIPE)])

  return edge_agg


_edge_agg = _make_edge_agg(NHID)

_ROWS = 1000
_GRID = N // _ROWS


def _mm_body(x_ref, w_ref, o_ref):
  o_ref[...] = jnp.dot(x_ref[...], w_ref[...],
                       preferred_element_type=jnp.float32)


def _tc_matmul(x, w):
  f, k = w.shape
  return pl.pallas_call(
      _mm_body,
      grid=(_GRID,),
      in_specs=[
          pl.BlockSpec((_ROWS, f), lambda i: (i, 0)),
          pl.BlockSpec((f, k), lambda i: (0, 0)),
      ],
      out_specs=pl.BlockSpec((_ROWS, k), lambda i: (i, 0)),
      out_shape=jax.ShapeDtypeStruct((N, k), jnp.float32),
  )(x, w)


def _combine_mm_body(p_ref, b_ref, w_ref, o_ref):
  h = jnp.maximum(p_ref[0] + p_ref[1] + b_ref[...], 0.0)
  o_ref[...] = jnp.dot(h, w_ref[...], preferred_element_type=jnp.float32)


def _tc_combine_matmul(p, b, w):
  f, k = w.shape
  return pl.pallas_call(
      _combine_mm_body,
      grid=(_GRID,),
      in_specs=[
          pl.BlockSpec((2, _ROWS, f), lambda i: (0, i, 0)),
          pl.BlockSpec((1, f), lambda i: (0, 0)),
          pl.BlockSpec((f, k), lambda i: (0, 0)),
      ],
      out_specs=pl.BlockSpec((_ROWS, k), lambda i: (i, 0)),
      out_shape=jax.ShapeDtypeStruct((N, k), jnp.float32),
  )(p, b, w)


def _final_body(p_ref, b_ref, o_ref):
  o_ref[...] = (p_ref[0, :, :NCLASS] + p_ref[1, :, :NCLASS] + b_ref[...])


def _tc_final(p, b):
  return pl.pallas_call(
      _final_body,
      grid=(_GRID,),
      in_specs=[
          pl.BlockSpec((2, _ROWS, NHID), lambda i: (0, i, 0)),
          pl.BlockSpec((1, NCLASS), lambda i: (0, 0)),
      ],
      out_specs=pl.BlockSpec((_ROWS, NCLASS), lambda i: (i, 0)),
      out_shape=jax.ShapeDtypeStruct((N, NCLASS), jnp.float32),
  )(p, b)


def _agg(s, src3, dst3, adj):
  return _edge_agg(s, src3, dst3, adj)[:, :N, :]


def kernel(x, adj, edge_index, isVal, W0, b0, W1, b1, W2, b2, Wo, bo):
  del isVal
  pad = _EPAD - E
  src3 = jnp.pad(edge_index[0], (0, pad)).reshape(32 * _NCHUNK, _CHUNK, _B)
  dst3 = jnp.pad(edge_index[1], (0, pad)).reshape(32 * _NCHUNK, _CHUNK, _B)
  adj = jnp.pad(adj, (0, pad))
  wo_pad = jnp.pad(Wo, ((0, 0), (0, NHID - NCLASS)))

  s = _tc_matmul(x, W0)
  p = _agg(s, src3, dst3, adj)
  s = _tc_combine_matmul(p, b0.reshape(1, NHID), W1)
  p = _agg(s, src3, dst3, adj)
  s = _tc_combine_matmul(p, b1.reshape(1, NHID), W2)
  p = _agg(s, src3, dst3, adj)
  s = _tc_combine_matmul(p, b2.reshape(1, NHID), wo_pad)
  p = _agg(s, src3, dst3, adj)
  return _tc_final(p, bo.reshape(1, NCLASS))


# speedup vs baseline: 1.7699x; 1.7699x over previous
import functools

import jax
import jax.numpy as jnp
from jax import lax
from jax.experimental import pallas as pl
from jax.experimental.pallas import tpu as pltpu
from jax.experimental.pallas import tpu_sc as plsc

N = 10000
E = 320000
NHID = 128
NCLASS = 64

_B = 80
_EPT = E // 32
_EPAD = E
_TPB = _EPT // _B
_CHUNK = 25
_NCHUNK = _TPB // _CHUNK
_NSLOT = 3
_NPAD = 10112
_STRIPE = _NPAD // 16

_DN = lax.GatherDimensionNumbers(
    offset_dims=(), collapsed_slice_dims=(0,), start_index_map=(0,))


def _make_edge_agg(d):
  mesh = plsc.VectorSubcoreMesh(core_axis_name="c", subcore_axis_name="s")

  @functools.partial(
      pl.kernel,
      mesh=mesh,
      out_type=jax.ShapeDtypeStruct((2, _NPAD, d), jnp.float32),
      scratch_types=[
          pltpu.VMEM((_CHUNK, _B), jnp.int32),
          pltpu.VMEM((_CHUNK, _B), jnp.int32),
          pltpu.VMEM((_CHUNK * _B,), jnp.float32),
          pltpu.VMEM((_NSLOT * _B, d), jnp.float32),
          pltpu.VMEM_SHARED((_NPAD, d), jnp.float32),
          pltpu.SemaphoreType.DMA,
          pltpu.SemaphoreType.DMA,
      ],
  )
  def edge_agg(s_hbm, src_hbm, dst_hbm, adj_hbm, out_hbm,
               src_v, dst_v, adj_v, ring_v, acc, gsem, ssem):
    c = lax.axis_index("c")
    sid = lax.axis_index("s")
    w = c * 16 + sid

    def _slot(t):
      return ring_v.at[pl.ds(pl.multiple_of(lax.rem(t, _NSLOT) * _B, 16), _B)]

    def zrow(i, carry):
      for k in range(d // 16):
        ring_v[i, pl.ds(k * 16, 16)] = jnp.zeros((16,), jnp.float32)
      return carry

    lax.fori_loop(0, 80, zrow, 0)
    zcopies = [(q * 80, 80) for q in range(_STRIPE // 80)]
    zcopies.append(((_STRIPE // 80) * 80, _STRIPE - (_STRIPE // 80) * 80))
    for off, n in zcopies:
      pltpu.async_copy(ring_v.at[pl.ds(0, n)],
                       acc.at[pl.ds(sid * _STRIPE + off, n)], gsem)
    for off, n in zcopies:
      pltpu.make_async_copy(ring_v.at[pl.ds(0, n)],
                            acc.at[pl.ds(sid * _STRIPE + off, n)], gsem).wait()
    plsc.subcore_barrier()

    def _load_idx(q):
      m = w * _NCHUNK + q
      pltpu.sync_copy(src_hbm.at[m], src_v)
      pltpu.sync_copy(dst_hbm.at[m], dst_v)
      pltpu.sync_copy(adj_hbm.at[pl.ds(m * (_CHUNK * _B), _CHUNK * _B)],
                      adj_v)

    def _gather(jj, i):
      pltpu.async_copy(s_hbm.at[src_v.at[jj]], _slot(i), gsem)

    def _wait_gather(jj, i):
      pltpu.make_async_copy(s_hbm.at[src_v.at[jj]], _slot(i), gsem).wait()

    def _scatter(jj, i):
      pltpu.async_copy(_slot(i), acc.at[dst_v.at[jj]], ssem, add=True)

    def _drain_scatter():
      pltpu.make_async_copy(ring_v.at[pl.ds(0, _B)], acc.at[dst_v.at[0]],
                            ssem).wait()

    def scale_batch(t, jj):
      base = pl.multiple_of(lax.rem(t, _NSLOT) * _B, 16)

      def scale(g, c2):
        va = adj_v[pl.ds(jj * _B + g * 16, 16)]
        for l in range(16):
          e = base + g * 16 + l
          a = lax.gather(va, jnp.full((16, 1), l, jnp.int32), _DN,
                         slice_sizes=(1,),
                         mode=lax.GatherScatterMode.PROMISE_IN_BOUNDS)
          for k in range(d // 16):
            ring_v[e, pl.ds(k * 16, 16)] = ring_v[e, pl.ds(k * 16, 16)] * a
        return c2

      lax.fori_loop(0, _B // 16, scale, 0)

    _load_idx(0)
    _gather(0, 0)
    _gather(1, 1)

    def chunk(q, carry):
      def batch(jj, carry2):
        t = q * _CHUNK + jj
        _wait_gather(jj, t)

        scale_batch(t, jj)
        _scatter(jj, t)

        @pl.when(jj >= 1)
        def _():
          _drain_scatter()

        @pl.when(jnp.logical_and(jj < _CHUNK - 2, t < _TPB - 2))
        def _():
          _gather(jj + 2, t + 2)
        return carry2

      lax.fori_loop(0, _CHUNK, batch, 0)
      _drain_scatter()

      @pl.when(q < _NCHUNK - 1)
      def _():
        _load_idx(q + 1)
        _gather(0, (q + 1) * _CHUNK)
        _gather(1, (q + 1) * _CHUNK + 1)

      return carry

    lax.fori_loop(0, _NCHUNK, chunk, 0)
    plsc.subcore_barrier()
    pltpu.sync_copy(acc.at[pl.ds(sid * _STRIPE, _STRIPE)],
                    out_hbm.at[c].at[pl.ds(sid * _STRIPE, _STRIPE)])

  return edge_agg


_edge_agg = _make_edge_agg(NHID)

_ROWS = 1000
_GRID = N // _ROWS


def _mm_body(x_ref, w_ref, o_ref):
  o_ref[...] = jnp.dot(x_ref[...], w_ref[...],
                       preferred_element_type=jnp.float32)


def _tc_matmul(x, w):
  f, k = w.shape
  return pl.pallas_call(
      _mm_body,
      grid=(_GRID,),
      in_specs=[
          pl.BlockSpec((_ROWS, f), lambda i: (i, 0)),
          pl.BlockSpec((f, k), lambda i: (0, 0)),
      ],
      out_specs=pl.BlockSpec((_ROWS, k), lambda i: (i, 0)),
      out_shape=jax.ShapeDtypeStruct((N, k), jnp.float32),
  )(x, w)


def _combine_mm_body(p_ref, b_ref, w_ref, o_ref):
  h = jnp.maximum(p_ref[0] + p_ref[1] + b_ref[...], 0.0)
  o_ref[...] = jnp.dot(h, w_ref[...], preferred_element_type=jnp.float32)


def _tc_combine_matmul(p, b, w):
  f, k = w.shape
  return pl.pallas_call(
      _combine_mm_body,
      grid=(_GRID,),
      in_specs=[
          pl.BlockSpec((2, _ROWS, f), lambda i: (0, i, 0)),
          pl.BlockSpec((1, f), lambda i: (0, 0)),
          pl.BlockSpec((f, k), lambda i: (0, 0)),
      ],
      out_specs=pl.BlockSpec((_ROWS, k), lambda i: (i, 0)),
      out_shape=jax.ShapeDtypeStruct((N, k), jnp.float32),
  )(p, b, w)


def _final_body(p_ref, b_ref, o_ref):
  o_ref[...] = (p_ref[0, :, :NCLASS] + p_ref[1, :, :NCLASS] + b_ref[...])


def _tc_final(p, b):
  return pl.pallas_call(
      _final_body,
      grid=(_GRID,),
      in_specs=[
          pl.BlockSpec((2, _ROWS, NHID), lambda i: (0, i, 0)),
          pl.BlockSpec((1, NCLASS), lambda i: (0, 0)),
      ],
      out_specs=pl.BlockSpec((_ROWS, NCLASS), lambda i: (i, 0)),
      out_shape=jax.ShapeDtypeStruct((N, NCLASS), jnp.float32),
  )(p, b)


def _agg(s, src3, dst3, adj):
  return _edge_agg(s, src3, dst3, adj)[:, :N, :]


def kernel(x, adj, edge_index, isVal, W0, b0, W1, b1, W2, b2, Wo, bo):
  del isVal
  src3 = edge_index[0].reshape(32 * _NCHUNK, _CHUNK, _B)
  dst3 = edge_index[1].reshape(32 * _NCHUNK, _CHUNK, _B)
  wo_pad = jnp.pad(Wo, ((0, 0), (0, NHID - NCLASS)))

  s = _tc_matmul(x, W0)
  p = _agg(s, src3, dst3, adj)
  s = _tc_combine_matmul(p, b0.reshape(1, NHID), W1)
  p = _agg(s, src3, dst3, adj)
  s = _tc_combine_matmul(p, b1.reshape(1, NHID), W2)
  p = _agg(s, src3, dst3, adj)
  s = _tc_combine_matmul(p, b2.reshape(1, NHID), wo_pad)
  p = _agg(s, src3, dst3, adj)
  return _tc_final(p, bo.reshape(1, NCLASS))


# speedup vs baseline: 1.8050x; 1.0198x over previous
import functools

import jax
import jax.numpy as jnp
from jax import lax
from jax.experimental import pallas as pl
from jax.experimental.pallas import tpu as pltpu
from jax.experimental.pallas import tpu_sc as plsc

N = 10000
E = 320000
NHID = 128
NCLASS = 64

_B = 80
_EPT = E // 32
_EPAD = E
_TPB = _EPT // _B
_CHUNK = 25
_NCHUNK = _TPB // _CHUNK
_NSLOT = 3
_NPAD = 10112
_STRIPE = _NPAD // 16

_DN = lax.GatherDimensionNumbers(
    offset_dims=(), collapsed_slice_dims=(0,), start_index_map=(0,))


def _make_edge_agg(d):
  mesh = plsc.VectorSubcoreMesh(core_axis_name="c", subcore_axis_name="s")

  @functools.partial(
      pl.kernel,
      mesh=mesh,
      out_type=jax.ShapeDtypeStruct((2, _NPAD, d), jnp.float32),
      scratch_types=[
          pltpu.VMEM((_CHUNK, _B), jnp.int32),
          pltpu.VMEM((_CHUNK, _B), jnp.int32),
          pltpu.VMEM((_CHUNK * _B,), jnp.float32),
          pltpu.VMEM((_NSLOT * _B, d), jnp.float32),
          pltpu.VMEM_SHARED((_NPAD, d), jnp.float32),
          pltpu.SemaphoreType.DMA,
          pltpu.SemaphoreType.DMA,
      ],
  )
  def edge_agg(s_hbm, src_hbm, dst_hbm, adj_hbm, out_hbm,
               src_v, dst_v, adj_v, ring_v, acc, gsem, ssem):
    c = lax.axis_index("c")
    sid = lax.axis_index("s")
    w = c * 16 + sid

    def _slot(t):
      return ring_v.at[pl.ds(pl.multiple_of(lax.rem(t, _NSLOT) * _B, 16), _B)]

    def zrow(i, carry):
      for k in range(d // 16):
        ring_v[i, pl.ds(k * 16, 16)] = jnp.zeros((16,), jnp.float32)
      return carry

    lax.fori_loop(0, 80, zrow, 0)
    zcopies = [(q * 80, 80) for q in range(_STRIPE // 80)]
    zcopies.append(((_STRIPE // 80) * 80, _STRIPE - (_STRIPE // 80) * 80))
    for off, n in zcopies:
      pltpu.async_copy(ring_v.at[pl.ds(0, n)],
                       acc.at[pl.ds(sid * _STRIPE + off, n)], gsem)
    for off, n in zcopies:
      pltpu.make_async_copy(ring_v.at[pl.ds(0, n)],
                            acc.at[pl.ds(sid * _STRIPE + off, n)], gsem).wait()
    plsc.subcore_barrier()

    def _load_idx(q):
      m = w * _NCHUNK + q
      pltpu.sync_copy(src_hbm.at[m], src_v)
      pltpu.sync_copy(dst_hbm.at[m], dst_v)
      pltpu.sync_copy(adj_hbm.at[pl.ds(m * (_CHUNK * _B), _CHUNK * _B)],
                      adj_v)

    def _gather(jj, i):
      pltpu.async_copy(s_hbm.at[src_v.at[jj]], _slot(i), gsem)

    def _wait_gather(jj, i):
      pltpu.make_async_copy(s_hbm.at[src_v.at[jj]], _slot(i), gsem).wait()

    def _scatter(jj, i):
      pltpu.async_copy(_slot(i), acc.at[dst_v.at[jj]], ssem, add=True)

    def _drain_scatter():
      pltpu.make_async_copy(ring_v.at[pl.ds(0, _B)], acc.at[dst_v.at[0]],
                            ssem).wait()

    def scale_batch(t, jj):
      base = pl.multiple_of(lax.rem(t, _NSLOT) * _B, 16)

      def scale(g, c2):
        va = adj_v[pl.ds(jj * _B + g * 16, 16)]
        for l in range(16):
          e = base + g * 16 + l
          a = lax.gather(va, jnp.full((16, 1), l, jnp.int32), _DN,
                         slice_sizes=(1,),
                         mode=lax.GatherScatterMode.PROMISE_IN_BOUNDS)
          for k in range(d // 16):
            ring_v[e, pl.ds(k * 16, 16)] = ring_v[e, pl.ds(k * 16, 16)] * a
        return c2

      lax.fori_loop(0, _B // 16, scale, 0)

    _load_idx(0)
    _gather(0, 0)
    _gather(1, 1)

    def chunk(q, carry):
      def batch(jj, carry2):
        t = q * _CHUNK + jj
        _wait_gather(jj, t)

        scale_batch(t, jj)
        _scatter(jj, t)

        @pl.when(jj >= 1)
        def _():
          _drain_scatter()

        @pl.when(jnp.logical_and(jj < _CHUNK - 2, t < _TPB - 2))
        def _():
          _gather(jj + 2, t + 2)
        return carry2

      lax.fori_loop(0, _CHUNK, batch, 0)
      _drain_scatter()

      @pl.when(q < _NCHUNK - 1)
      def _():
        _load_idx(q + 1)
        _gather(0, (q + 1) * _CHUNK)
        _gather(1, (q + 1) * _CHUNK + 1)

      return carry

    lax.fori_loop(0, _NCHUNK, chunk, 0)
    plsc.subcore_barrier()
    pltpu.sync_copy(acc.at[pl.ds(sid * _STRIPE, _STRIPE)],
                    out_hbm.at[c].at[pl.ds(sid * _STRIPE, _STRIPE)])

  return edge_agg


_edge_agg = _make_edge_agg(NHID)

_ROWS = 2000
_GRID = N // _ROWS


def _mm_body(x_ref, w_ref, o_ref):
  o_ref[...] = jnp.dot(x_ref[...], w_ref[...],
                       preferred_element_type=jnp.float32)


def _tc_matmul(x, w):
  f, k = w.shape
  return pl.pallas_call(
      _mm_body,
      grid=(_GRID,),
      in_specs=[
          pl.BlockSpec((_ROWS, f), lambda i: (i, 0)),
          pl.BlockSpec((f, k), lambda i: (0, 0)),
      ],
      out_specs=pl.BlockSpec((_ROWS, k), lambda i: (i, 0)),
      out_shape=jax.ShapeDtypeStruct((N, k), jnp.float32),
  )(x, w)


def _combine_mm_body(p_ref, b_ref, w_ref, o_ref):
  h = jnp.maximum(p_ref[0] + p_ref[1] + b_ref[...], 0.0)
  o_ref[...] = jnp.dot(h, w_ref[...], preferred_element_type=jnp.float32)


def _tc_combine_matmul(p, b, w):
  f, k = w.shape
  return pl.pallas_call(
      _combine_mm_body,
      grid=(_GRID,),
      in_specs=[
          pl.BlockSpec((2, _ROWS, f), lambda i: (0, i, 0)),
          pl.BlockSpec((1, f), lambda i: (0, 0)),
          pl.BlockSpec((f, k), lambda i: (0, 0)),
      ],
      out_specs=pl.BlockSpec((_ROWS, k), lambda i: (i, 0)),
      out_shape=jax.ShapeDtypeStruct((N, k), jnp.float32),
  )(p, b, w)


def _final_body(p_ref, b_ref, o_ref):
  o_ref[...] = (p_ref[0, :, :NCLASS] + p_ref[1, :, :NCLASS] + b_ref[...])


def _tc_final(p, b):
  return pl.pallas_call(
      _final_body,
      grid=(_GRID,),
      in_specs=[
          pl.BlockSpec((2, _ROWS, NHID), lambda i: (0, i, 0)),
          pl.BlockSpec((1, NCLASS), lambda i: (0, 0)),
      ],
      out_specs=pl.BlockSpec((_ROWS, NCLASS), lambda i: (i, 0)),
      out_shape=jax.ShapeDtypeStruct((N, NCLASS), jnp.float32),
  )(p, b)


def _agg(s, src3, dst3, adj):
  return _edge_agg(s, src3, dst3, adj)[:, :N, :]


def kernel(x, adj, edge_index, isVal, W0, b0, W1, b1, W2, b2, Wo, bo):
  del isVal
  src3 = edge_index[0].reshape(32 * _NCHUNK, _CHUNK, _B)
  dst3 = edge_index[1].reshape(32 * _NCHUNK, _CHUNK, _B)
  wo_pad = jnp.pad(Wo, ((0, 0), (0, NHID - NCLASS)))

  s = _tc_matmul(x, W0)
  p = _agg(s, src3, dst3, adj)
  s = _tc_combine_matmul(p, b0.reshape(1, NHID), W1)
  p = _agg(s, src3, dst3, adj)
  s = _tc_combine_matmul(p, b1.reshape(1, NHID), W2)
  p = _agg(s, src3, dst3, adj)
  s = _tc_combine_matmul(p, b2.reshape(1, NHID), wo_pad)
  p = _agg(s, src3, dst3, adj)
  return _tc_final(p, bo.reshape(1, NCLASS))
